# Initial kernel scaffold; baseline (speedup 1.0000x reference)
import jax
import jax.numpy as jnp
from jax.experimental import pallas as pl


def kernel(x, edge_index, params):
    raise NotImplementedError("write your pallas kernel here")


# speedup vs baseline: 2.2663x; 2.2663x over previous
import functools

import jax
import jax.numpy as jnp
from jax import lax
from jax.experimental import pallas as pl
from jax.experimental.pallas import tpu as pltpu
from jax.experimental.pallas import tpu_sc as plsc

N = 10000
E = 160000
F = 128
EROWS = 1280
EPAD = EROWS * F
NSUB = 16
ROWS_PER_TILE = EROWS // NSUB
NJ = N + 16
BN = 1000


GRP = 8
GROUPS = ROWS_PER_TILE // GRP
NSPLIT = 624


@functools.lru_cache(maxsize=None)
def _segsum(nchunks):
    mesh = plsc.VectorSubcoreMesh(core_axis_name="c", subcore_axis_name="s",
                                  num_cores=2, num_subcores=NSUB)
    out_type = tuple(jax.ShapeDtypeStruct((N, F), jnp.float32)
                     for _ in range(nchunks))
    scratch = [
        pltpu.VMEM((GRP, F), jnp.int32),
        pltpu.VMEM((GRP, F), jnp.int32),
        pltpu.VMEM((F, F), jnp.float32),
        pltpu.VMEM_SHARED((NJ, F), jnp.float32),
        pltpu.SemaphoreType.DMA,
    ]

    def body(*refs):
        hs = refs[:nchunks]
        src_r, dst_r, zeros_r = refs[nchunks:nchunks + 3]
        aggs = refs[nchunks + 3:2 * nchunks + 3]
        srcb, dstb, gbuf, accum, sem = refs[2 * nchunks + 3:]
        c = lax.axis_index("c")
        s = lax.axis_index("s")

        def run_chunk(h_ref, agg_ref):
            zoff = pl.multiple_of(s * NSPLIT, 8)
            pltpu.sync_copy(zeros_r.at[pl.ds(zoff, NSPLIT)],
                            accum.at[pl.ds(zoff, NSPLIT)])

            @pl.when(s == 0)
            def _():
                tail = pl.ds(NSUB * NSPLIT, NJ - NSUB * NSPLIT)
                pltpu.sync_copy(zeros_r.at[tail], accum.at[tail])

            plsc.subcore_barrier()

            def step(g, carry):
                grp = pl.multiple_of(s * ROWS_PER_TILE + g * GRP, 8)
                pltpu.sync_copy(src_r.at[pl.ds(grp, GRP)], srcb)
                pltpu.sync_copy(dst_r.at[pl.ds(grp, GRP)], dstb)
                for j in range(GRP):
                    pltpu.async_copy(h_ref.at[srcb.at[j]], gbuf, sem).wait()
                    pltpu.sync_copy(gbuf, accum.at[dstb.at[j]], add=True)
                return carry

            lax.fori_loop(0, GROUPS, step, 0)
            plsc.subcore_barrier()
            ooff = pl.multiple_of(s * NSPLIT, 8)
            pltpu.sync_copy(accum.at[pl.ds(ooff, NSPLIT)],
                            agg_ref.at[pl.ds(ooff, NSPLIT)])

            @pl.when(s == 0)
            def _():
                tail = pl.ds(NSUB * NSPLIT, N - NSUB * NSPLIT)
                pltpu.sync_copy(accum.at[tail], agg_ref.at[tail])

            plsc.subcore_barrier()

        half = nchunks // 2

        @pl.when(c == 0)
        def _():
            for q in range(half):
                run_chunk(hs[q], aggs[q])

        @pl.when(c == 1)
        def _():
            for q in range(half, nchunks):
                run_chunk(hs[q], aggs[q])

    return pl.kernel(body, out_type=out_type, mesh=mesh,
                     scratch_types=scratch, name=f"gin_segsum{nchunks}")


def _ln_relu_mlp(z, w1, b1, w2, b2, g, bt):
    t = jnp.maximum(jnp.dot(z, w1, preferred_element_type=jnp.float32) + b1, 0.0)
    t = jnp.dot(t, w2, preferred_element_type=jnp.float32) + b2
    mu = jnp.mean(t, axis=-1, keepdims=True)
    d = t - mu
    var = jnp.mean(d * d, axis=-1, keepdims=True)
    t = d * lax.rsqrt(var + 1e-5) * g + bt
    return jnp.maximum(t, 0.0)


@functools.lru_cache(maxsize=None)
def _mlp_hidden(nc_in, in_dim):
    def body(*refs):
        hs = refs[:nc_in]
        ags = refs[nc_in:2 * nc_in]
        w1, b1, w2, b2, g, bt = refs[2 * nc_in:2 * nc_in + 6]
        outs = refs[2 * nc_in + 6:]
        z = jnp.concatenate([hs[i][...] + ags[i][...] for i in range(nc_in)],
                            axis=-1)
        hn = _ln_relu_mlp(z, w1[...], b1[...], w2[...], b2[...], g[...], bt[...])
        for q in range(4):
            outs[q][...] = hn[:, q * F:(q + 1) * F]

    blk = pl.BlockSpec((BN, F), lambda i: (i, 0))
    full = lambda shape: pl.BlockSpec(shape, lambda i: (0, 0))
    in_specs = ([blk] * (2 * nc_in)
                + [full((in_dim, 512)), full((1, 512)), full((512, 512)),
                   full((1, 512)), full((1, 512)), full((1, 512))])
    return pl.pallas_call(
        body,
        grid=(N // BN,),
        in_specs=in_specs,
        out_specs=[blk] * 4,
        out_shape=[jax.ShapeDtypeStruct((N, F), jnp.float32)] * 4,
        name="gin_mlp",
    )


@functools.lru_cache(maxsize=None)
def _mlp_final(nc_in):
    def body(*refs):
        hs = refs[:nc_in]
        ags = refs[nc_in:2 * nc_in]
        w1, b1, w2, b2, g, bt, wc1, bc1, wc2r, bc2 = refs[2 * nc_in:2 * nc_in + 10]
        out, = refs[2 * nc_in + 10:]
        z = jnp.concatenate([hs[i][...] + ags[i][...] for i in range(nc_in)],
                            axis=-1)
        hn = _ln_relu_mlp(z, w1[...], b1[...], w2[...], b2[...], g[...], bt[...])
        u = jnp.maximum(jnp.dot(hn, wc1[...], preferred_element_type=jnp.float32)
                        + bc1[...], 0.0)
        out[...] = (jnp.sum(u * wc2r[...], axis=-1, keepdims=True) + bc2[...])

    blk = pl.BlockSpec((BN, F), lambda i: (i, 0))
    full = lambda shape: pl.BlockSpec(shape, lambda i: (0, 0))
    in_specs = ([blk] * (2 * nc_in)
                + [full((512, 512)), full((1, 512)), full((512, 512)),
                   full((1, 512)), full((1, 512)), full((1, 512)),
                   full((512, 512)), full((1, 512)), full((1, 512)),
                   full((1, 1))])
    return pl.pallas_call(
        body,
        grid=(N // BN,),
        in_specs=in_specs,
        out_specs=pl.BlockSpec((BN, 1), lambda i: (i, 0)),
        out_shape=jax.ShapeDtypeStruct((N, 1), jnp.float32),
        name="gin_mlp_final",
    )


def kernel(x, edge_index, params):
    src = edge_index[0]
    dst = edge_index[1]
    pad = EPAD - E
    src_p = jnp.concatenate([src, jnp.zeros((pad,), jnp.int32)])
    src_p = src_p.reshape(EROWS, F)
    dst_p = jnp.concatenate([dst, jnp.full((pad,), N, jnp.int32)])
    dst_p = dst_p.reshape(EROWS, F)
    zeros = jnp.zeros((NJ, F), jnp.float32)

    hs = [x[:, :F], x[:, F:]]
    out = None
    for li, p in enumerate(params["layers"]):
        nc = len(hs)
        aggs = _segsum(nc)(*hs, src_p, dst_p, zeros)
        w1 = p["W1"]
        b1 = p["b1"].reshape(1, -1)
        w2 = p["W2"]
        b2 = p["b2"].reshape(1, -1)
        g = p["gamma"].reshape(1, -1)
        bt = p["beta"].reshape(1, -1)
        if li < 3:
            hs = list(_mlp_hidden(nc, w1.shape[0])(*hs, *aggs, w1, b1, w2, b2,
                                                   g, bt))
        else:
            out = _mlp_final(nc)(
                *hs, *aggs, w1, b1, w2, b2, g, bt,
                params["Wc1"], params["bc1"].reshape(1, -1),
                params["Wc2"].reshape(1, -1), params["bc2"].reshape(1, 1))
    return out


# speedup vs baseline: 2.6951x; 1.1892x over previous
import functools

import jax
import jax.numpy as jnp
from jax import lax
from jax.experimental import pallas as pl
from jax.experimental.pallas import tpu as pltpu
from jax.experimental.pallas import tpu_sc as plsc

N = 10000
E = 160000
F = 128
EROWS = 1280
EPAD = EROWS * F
NSUB = 16
ROWS_PER_TILE = EROWS // NSUB
NJ = N + 16
BN = 1000


GRP = 8
GROUPS = ROWS_PER_TILE // GRP
NSPLIT = 624


@functools.lru_cache(maxsize=None)
def _segsum(nchunks):
    mesh = plsc.VectorSubcoreMesh(core_axis_name="c", subcore_axis_name="s",
                                  num_cores=2, num_subcores=NSUB)
    out_type = tuple(jax.ShapeDtypeStruct((N, F), jnp.float32)
                     for _ in range(nchunks))
    scratch = [
        pltpu.VMEM((2, GRP, F), jnp.int32),
        pltpu.VMEM((2, GRP, F), jnp.int32),
        pltpu.VMEM((2, F, F), jnp.float32),
        pltpu.VMEM_SHARED((NJ, F), jnp.float32),
        pltpu.SemaphoreType.DMA,
        pltpu.SemaphoreType.DMA,
        pltpu.SemaphoreType.DMA,
    ]

    def body(*refs):
        hs = refs[:nchunks]
        src_r, dst_r, zeros_r = refs[nchunks:nchunks + 3]
        aggs = refs[nchunks + 3:2 * nchunks + 3]
        srcb, dstb, gbuf, accum, sem0, sem1, semi = refs[2 * nchunks + 3:]
        gsem = (sem0, sem1)
        c = lax.axis_index("c")
        s = lax.axis_index("s")
        base = s * ROWS_PER_TILE

        def grp_slice(g):
            return pl.ds(pl.multiple_of(base + g * GRP, 8), GRP)

        def run_chunk(h_ref, agg_ref):
            zoff = pl.multiple_of(s * NSPLIT, 8)
            pltpu.sync_copy(zeros_r.at[pl.ds(zoff, NSPLIT)],
                            accum.at[pl.ds(zoff, NSPLIT)])

            @pl.when(s == 0)
            def _():
                tail = pl.ds(NSUB * NSPLIT, NJ - NSUB * NSPLIT)
                pltpu.sync_copy(zeros_r.at[tail], accum.at[tail])

            plsc.subcore_barrier()

            pltpu.sync_copy(src_r.at[grp_slice(0)], srcb.at[0])
            pltpu.sync_copy(dst_r.at[grp_slice(0)], dstb.at[0])
            pltpu.async_copy(h_ref.at[srcb.at[0].at[0]], gbuf.at[0], sem0)

            def step(g, carry):
                p = g % 2

                @pl.when(g > 0)
                def _():
                    pltpu.make_async_copy(src_r.at[grp_slice(g)], srcb.at[p],
                                          semi).wait()
                    pltpu.make_async_copy(dst_r.at[grp_slice(g)], dstb.at[p],
                                          semi).wait()
                    pltpu.async_copy(h_ref.at[srcb.at[p].at[0]], gbuf.at[0],
                                     sem0)

                @pl.when(g < GROUPS - 1)
                def _():
                    pltpu.async_copy(src_r.at[grp_slice(g + 1)],
                                     srcb.at[1 - p], semi)
                    pltpu.async_copy(dst_r.at[grp_slice(g + 1)],
                                     dstb.at[1 - p], semi)

                for j in range(GRP):
                    bj = j % 2
                    if j < GRP - 1:
                        pltpu.async_copy(h_ref.at[srcb.at[p].at[j + 1]],
                                         gbuf.at[1 - bj], gsem[1 - bj])
                    pltpu.make_async_copy(h_ref.at[srcb.at[p].at[j]],
                                          gbuf.at[bj], gsem[bj]).wait()
                    pltpu.sync_copy(gbuf.at[bj], accum.at[dstb.at[p].at[j]],
                                    add=True)
                return carry

            lax.fori_loop(0, GROUPS, step, 0)
            plsc.subcore_barrier()
            ooff = pl.multiple_of(s * NSPLIT, 8)
            pltpu.sync_copy(accum.at[pl.ds(ooff, NSPLIT)],
                            agg_ref.at[pl.ds(ooff, NSPLIT)])

            @pl.when(s == 0)
            def _():
                tail = pl.ds(NSUB * NSPLIT, N - NSUB * NSPLIT)
                pltpu.sync_copy(accum.at[tail], agg_ref.at[tail])

            plsc.subcore_barrier()

        half = nchunks // 2

        @pl.when(c == 0)
        def _():
            for q in range(half):
                run_chunk(hs[q], aggs[q])

        @pl.when(c == 1)
        def _():
            for q in range(half, nchunks):
                run_chunk(hs[q], aggs[q])

    return pl.kernel(body, out_type=out_type, mesh=mesh,
                     scratch_types=scratch, name=f"gin_segsum{nchunks}")


def _ln_relu_mlp(z, w1, b1, w2, b2, g, bt):
    t = jnp.maximum(jnp.dot(z, w1, preferred_element_type=jnp.float32) + b1, 0.0)
    t = jnp.dot(t, w2, preferred_element_type=jnp.float32) + b2
    mu = jnp.mean(t, axis=-1, keepdims=True)
    d = t - mu
    var = jnp.mean(d * d, axis=-1, keepdims=True)
    t = d * lax.rsqrt(var + 1e-5) * g + bt
    return jnp.maximum(t, 0.0)


@functools.lru_cache(maxsize=None)
def _mlp_hidden(nc_in, in_dim):
    def body(*refs):
        hs = refs[:nc_in]
        ags = refs[nc_in:2 * nc_in]
        w1, b1, w2, b2, g, bt = refs[2 * nc_in:2 * nc_in + 6]
        outs = refs[2 * nc_in + 6:]
        z = jnp.concatenate([hs[i][...] + ags[i][...] for i in range(nc_in)],
                            axis=-1)
        hn = _ln_relu_mlp(z, w1[...], b1[...], w2[...], b2[...], g[...], bt[...])
        for q in range(4):
            outs[q][...] = hn[:, q * F:(q + 1) * F]

    blk = pl.BlockSpec((BN, F), lambda i: (i, 0))
    full = lambda shape: pl.BlockSpec(shape, lambda i: (0, 0))
    in_specs = ([blk] * (2 * nc_in)
                + [full((in_dim, 512)), full((1, 512)), full((512, 512)),
                   full((1, 512)), full((1, 512)), full((1, 512))])
    return pl.pallas_call(
        body,
        grid=(N // BN,),
        in_specs=in_specs,
        out_specs=[blk] * 4,
        out_shape=[jax.ShapeDtypeStruct((N, F), jnp.float32)] * 4,
        name="gin_mlp",
    )


@functools.lru_cache(maxsize=None)
def _mlp_final(nc_in):
    def body(*refs):
        hs = refs[:nc_in]
        ags = refs[nc_in:2 * nc_in]
        w1, b1, w2, b2, g, bt, wc1, bc1, wc2r, bc2 = refs[2 * nc_in:2 * nc_in + 10]
        out, = refs[2 * nc_in + 10:]
        z = jnp.concatenate([hs[i][...] + ags[i][...] for i in range(nc_in)],
                            axis=-1)
        hn = _ln_relu_mlp(z, w1[...], b1[...], w2[...], b2[...], g[...], bt[...])
        u = jnp.maximum(jnp.dot(hn, wc1[...], preferred_element_type=jnp.float32)
                        + bc1[...], 0.0)
        out[...] = (jnp.sum(u * wc2r[...], axis=-1, keepdims=True) + bc2[...])

    blk = pl.BlockSpec((BN, F), lambda i: (i, 0))
    full = lambda shape: pl.BlockSpec(shape, lambda i: (0, 0))
    in_specs = ([blk] * (2 * nc_in)
                + [full((512, 512)), full((1, 512)), full((512, 512)),
                   full((1, 512)), full((1, 512)), full((1, 512)),
                   full((512, 512)), full((1, 512)), full((1, 512)),
                   full((1, 1))])
    return pl.pallas_call(
        body,
        grid=(N // BN,),
        in_specs=in_specs,
        out_specs=pl.BlockSpec((BN, 1), lambda i: (i, 0)),
        out_shape=jax.ShapeDtypeStruct((N, 1), jnp.float32),
        name="gin_mlp_final",
    )


def kernel(x, edge_index, params):
    src = edge_index[0]
    dst = edge_index[1]
    pad = EPAD - E
    src_p = jnp.concatenate([src, jnp.zeros((pad,), jnp.int32)])
    src_p = src_p.reshape(EROWS, F)
    dst_p = jnp.concatenate([dst, jnp.full((pad,), N, jnp.int32)])
    dst_p = dst_p.reshape(EROWS, F)
    zeros = jnp.zeros((NJ, F), jnp.float32)

    hs = [x[:, :F], x[:, F:]]
    out = None
    for li, p in enumerate(params["layers"]):
        nc = len(hs)
        aggs = _segsum(nc)(*hs, src_p, dst_p, zeros)
        w1 = p["W1"]
        b1 = p["b1"].reshape(1, -1)
        w2 = p["W2"]
        b2 = p["b2"].reshape(1, -1)
        g = p["gamma"].reshape(1, -1)
        bt = p["beta"].reshape(1, -1)
        if li < 3:
            hs = list(_mlp_hidden(nc, w1.shape[0])(*hs, *aggs, w1, b1, w2, b2,
                                                   g, bt))
        else:
            out = _mlp_final(nc)(
                *hs, *aggs, w1, b1, w2, b2, g, bt,
                params["Wc1"], params["bc1"].reshape(1, -1),
                params["Wc2"].reshape(1, -1), params["bc2"].reshape(1, 1))
    return out
